# baseline (speedup 1.0000x reference)
import jax
import jax.numpy as jnp
from jax import lax
from jax.experimental import pallas as pl
from jax.experimental.pallas import tpu as pltpu
from jax.experimental.pallas import tpu_sc as plsc

N = 4096
F = 256
E = 131072
NC = 2
NS = 16
NW = NC * NS
EPW = E // NW
EB = 128
NB = EPW // EB
RPT = N // NS

_f32 = jnp.float32
_MESH = plsc.VectorSubcoreMesh(
    core_axis_name="c", subcore_axis_name="s", num_cores=NC, num_subcores=NS)


def _prep_body(data_hbm, src_hbm, dst2_hbm,
               ew_hbm, degp_hbm,
               src_v, dst2_v, idx_v, ew_v, z_v, deg_sh, sem):
  cid = lax.axis_index("c")
  sid = lax.axis_index("s")
  wid = sid * NC + cid
  ebase = wid * EPW
  pltpu.sync_copy(src_hbm.at[pl.ds(ebase, EPW)], src_v)
  pltpu.sync_copy(dst2_hbm.at[pl.ds(wid * NB, NB)], dst2_v)

  def idx_body(b, c):
    for j in range(EB // 16):
      s16 = src_v[pl.ds(b * EB + j * 16, 16)]
      d16 = dst2_v[b, pl.ds(j * 16, 16)]
      idx_v[pl.ds(b * EB + j * 16, 16)] = s16 * N + d16
    return c
  lax.fori_loop(0, NB, idx_body, 0)

  cps = []
  for b in range(NB):
    cps.append(pltpu.async_copy(
        data_hbm.at[idx_v.at[pl.ds(b * EB, EB)]],
        ew_v.at[pl.ds(b * EB, EB)], sem))
  for cp in cps:
    cp.wait()
  pltpu.sync_copy(ew_v, ew_hbm.at[pl.ds(ebase, EPW)])

  for k in range(RPT // 16):
    z_v[pl.ds(k * 16, 16)] = jnp.zeros((16,), _f32)
  pltpu.sync_copy(z_v, deg_sh.at[pl.ds(sid * RPT, RPT)])
  plsc.subcore_barrier()
  for b in range(NB):
    pltpu.sync_copy(ew_v.at[pl.ds(b * EB, EB)],
                    deg_sh.at[dst2_v.at[b]], add=True)
  plsc.subcore_barrier()
  pltpu.sync_copy(deg_sh.at[pl.ds(sid * RPT, RPT)],
                  degp_hbm.at[pl.ds(cid * N + sid * RPT, RPT)])


def _prep(data_flat, src, dst2):
  return pl.kernel(
      _prep_body,
      out_type=(jax.ShapeDtypeStruct((E,), _f32),
                jax.ShapeDtypeStruct((NC * N,), _f32)),
      mesh=_MESH,
      scratch_types=[
          pltpu.VMEM((EPW,), jnp.int32),
          pltpu.VMEM((NB, EB), jnp.int32),
          pltpu.VMEM((EPW,), jnp.int32),
          pltpu.VMEM((EPW,), _f32),
          pltpu.VMEM((RPT,), _f32),
          pltpu.VMEM_SHARED((N,), _f32),
          pltpu.SemaphoreType.DMA,
      ],
  )(data_flat, src, dst2)


EPT = E // NS
NB2 = EPT // EB
SR = 256
NSLAB = N // SR // NC
DUMP = SR * N
ZW = 4096


CAP = 768


def _abuild(src, dst2, ew):
  def body(src_hbm, dst2_hbm, ew_hbm, a_hbm,
           src_v, dst2_v, ew_v, gidx_v, idx_v, zb_v, slab_sh, sem):
    cid = lax.axis_index("c")
    sid = lax.axis_index("s")
    ebase = sid * EPT
    pltpu.sync_copy(src_hbm.at[pl.ds(ebase, EPT)], src_v)
    pltpu.sync_copy(dst2_hbm.at[pl.ds(sid * NB2, NB2)], dst2_v)
    pltpu.sync_copy(ew_hbm.at[pl.ds(ebase, EPT)], ew_v)

    def zb_body(i, c):
      zb_v[pl.ds(i * 16, 16)] = jnp.zeros((16,), _f32)
      return c
    lax.fori_loop(0, ZW // 16, zb_body, 0)

    base16 = jnp.full((16,), 1, jnp.int32) * (cid * NSLAB)

    def pre_body(b, c):
      for j in range(EB // 16):
        s16 = src_v[pl.ds(b * EB + j * 16, 16)]
        d16 = dst2_v[b, pl.ds(j * 16, 16)]
        gidx_v[pl.ds(b * EB + j * 16, 16)] = (
            (d16 & jnp.full((16,), SR - 1, jnp.int32)) * N + s16)
      return c
    lax.fori_loop(0, NB2, pre_body, 0)

    dump16 = jnp.full((16,), DUMP, jnp.int32)
    for slab in range(NSLAB):
      lo = (cid * NSLAB + slab) * SR
      slab16 = jnp.full((16,), slab, jnp.int32)
      for k in range(SR * N // NS // ZW):
        pltpu.sync_copy(zb_v, slab_sh.at[pl.ds(sid * (SR * N // NS)
                                               + k * ZW, ZW)])
      plsc.subcore_barrier()

      def idx_body(b, c):
        for j in range(EB // 16):
          g16 = gidx_v[pl.ds(b * EB + j * 16, 16)]
          d16 = dst2_v[b, pl.ds(j * 16, 16)]
          ok = (lax.shift_right_logical(d16, 8) - base16) == slab16
          idx_v[b, pl.ds(j * 16, 16)] = jnp.where(ok, g16, dump16)
        return c
      lax.fori_loop(0, NB2, idx_body, 0)
      for b in range(NB2):
        pltpu.sync_copy(ew_v.at[pl.ds(b * EB, EB)],
                        slab_sh.at[idx_v.at[b]], add=True)
      plsc.subcore_barrier()
      pltpu.sync_copy(
          slab_sh.at[pl.ds(sid * (SR // NS) * N, (SR // NS) * N)],
          a_hbm.at[pl.ds((lo + sid * (SR // NS)) * N, (SR // NS) * N)])
      plsc.subcore_barrier()

  return pl.kernel(
      body,
      out_type=jax.ShapeDtypeStruct((N * N,), _f32),
      mesh=_MESH,
      scratch_types=[
          pltpu.VMEM((EPT,), jnp.int32),
          pltpu.VMEM((NB2, EB), jnp.int32),
          pltpu.VMEM((EPT,), _f32),
          pltpu.VMEM((EPT,), jnp.int32),
          pltpu.VMEM((NB2, EB), jnp.int32),
          pltpu.VMEM((ZW,), _f32),
          pltpu.VMEM_SHARED((SR * N + 8,), _f32),
          pltpu.SemaphoreType.DMA,
      ],
  )(src, dst2, ew)


def _dis_kernel(degm, degd):
  R = N // 128

  def body(dm_ref, dd_ref, om_ref, od_ref):
    ones = jnp.ones((1, 128), _f32)
    for dref, oref in ((dm_ref, om_ref), (dd_ref, od_ref)):
      d = dref[0:R] + dref[R:2 * R] + 1.0
      ok = d > 0
      ds_ = jnp.where(ok, d, 1.0)
      y = lax.rsqrt(ds_)
      y = 0.5 * y * (3.0 - ds_ * y * y)
      y = 0.5 * y * (3.0 - ds_ * y * y)
      dis = jnp.where(ok, y, 0.0)
      for i in range(R):
        oref[pl.ds(i * 128, 128), :] = lax.dot_general(
            dis[i:i + 1, :], ones, (((0,), (0,)), ((), ())),
            preferred_element_type=_f32, precision=lax.Precision.HIGHEST)
  out = pl.pallas_call(
      body,
      out_shape=[jax.ShapeDtypeStruct((N, 128), _f32),
                 jax.ShapeDtypeStruct((N, 128), _f32)],
  )(degm.reshape(2 * R, 128), degd.reshape(2 * R, 128))
  return out


_BR = 512


def _t1(x, W, dis2d):
  def body(x_ref, w_ref, dis_ref, o_ref):
    xw = jnp.dot(x_ref[...], w_ref[...], preferred_element_type=_f32, precision=lax.Precision.HIGHEST)
    o_ref[...] = dis_ref[:, 0:1] * xw
  return pl.pallas_call(
      body,
      grid=(N // _BR,),
      in_specs=[pl.BlockSpec((_BR, F), lambda i: (i, 0)),
                pl.BlockSpec((F, F), lambda i: (0, 0)),
                pl.BlockSpec((_BR, 128), lambda i: (i, 0))],
      out_specs=pl.BlockSpec((_BR, F), lambda i: (i, 0)),
      out_shape=jax.ShapeDtypeStruct((N, F), _f32),
  )(x, W, dis2d)


def _g1(A2d, u, dis2d, b1, W2):
  def body(a_ref, u_ref, dis_ref, b_ref, w_ref, o_ref):
    i = pl.program_id(0)
    acc = jnp.dot(a_ref[...], u_ref[...], preferred_element_type=_f32, precision=lax.Precision.HIGHEST)
    u_blk = u_ref[pl.ds(i * _BR, _BR), :]
    dis = dis_ref[:, 0:1]
    h = jax.nn.relu(dis * (acc + u_blk) + b_ref[...])
    o_ref[...] = dis * jnp.dot(h, w_ref[...], preferred_element_type=_f32, precision=lax.Precision.HIGHEST)
  return pl.pallas_call(
      body,
      grid=(N // _BR,),
      in_specs=[pl.BlockSpec((_BR, N), lambda i: (i, 0)),
                pl.BlockSpec((N, F), lambda i: (0, 0)),
                pl.BlockSpec((_BR, 128), lambda i: (i, 0)),
                pl.BlockSpec((1, F), lambda i: (0, 0)),
                pl.BlockSpec((F, F), lambda i: (0, 0))],
      out_specs=pl.BlockSpec((_BR, F), lambda i: (i, 0)),
      out_shape=jax.ShapeDtypeStruct((N, F), _f32),
  )(A2d, u, dis2d, b1, W2)


def _g2(A2d, u, dis2d, b2, L1, bL1, L2, bL2, L3, bL3):
  def body(a_ref, u_ref, dis_ref, b_ref,
           l1_ref, b1_ref, l2_ref, b2_ref, l3_ref, b3_ref, o_ref):
    i = pl.program_id(0)
    acc = jnp.dot(a_ref[...], u_ref[...], preferred_element_type=_f32, precision=lax.Precision.HIGHEST)
    u_blk = u_ref[pl.ds(i * _BR, _BR), :]
    dis = dis_ref[:, 0:1]
    X = jax.nn.relu(dis * (acc + u_blk) + b_ref[...])
    x1 = jax.nn.relu(jnp.dot(X, l1_ref[...], preferred_element_type=_f32, precision=lax.Precision.HIGHEST)
                     + b1_ref[...])
    x2 = jax.nn.relu(jnp.dot(x1, l2_ref[...], preferred_element_type=_f32, precision=lax.Precision.HIGHEST)
                     + b2_ref[...])
    o_ref[...] = jax.nn.relu(
        jnp.dot(x2, l3_ref[...], preferred_element_type=_f32, precision=lax.Precision.HIGHEST) + b3_ref[...])
  return pl.pallas_call(
      body,
      grid=(N // _BR,),
      in_specs=[pl.BlockSpec((_BR, N), lambda i: (i, 0)),
                pl.BlockSpec((N, F), lambda i: (0, 0)),
                pl.BlockSpec((_BR, 128), lambda i: (i, 0)),
                pl.BlockSpec((1, F), lambda i: (0, 0)),
                pl.BlockSpec((F, F), lambda i: (0, 0)),
                pl.BlockSpec((1, F), lambda i: (0, 0)),
                pl.BlockSpec((F, 128), lambda i: (0, 0)),
                pl.BlockSpec((1, 128), lambda i: (0, 0)),
                pl.BlockSpec((128, 64), lambda i: (0, 0)),
                pl.BlockSpec((1, 64), lambda i: (0, 0))],
      out_specs=pl.BlockSpec((_BR, 64), lambda i: (i, 0)),
      out_shape=jax.ShapeDtypeStruct((N, 64), _f32),
  )(A2d, u, dis2d, b2, L1, bL1, L2, bL2, L3, bL3)


def _final(a, b):
  def body(a_ref, b_ref, o_ref):
    o_ref[...] = lax.dot_general(a_ref[...], b_ref[...],
                                 (((1,), (1,)), ((), ())),
                                 preferred_element_type=_f32, precision=lax.Precision.HIGHEST)
  return pl.pallas_call(
      body,
      grid=(N // _BR, N // _BR),
      in_specs=[pl.BlockSpec((_BR, 64), lambda i, j: (i, 0)),
                pl.BlockSpec((_BR, 64), lambda i, j: (j, 0))],
      out_specs=pl.BlockSpec((_BR, _BR), lambda i, j: (i, j)),
      out_shape=jax.ShapeDtypeStruct((N, N), _f32),
  )(a, b)


def kernel(x_m, x_d, data_m, data_d, edge_index_m, edge_index_d,
           Wx1, bx1, Wx2, bx2, Wy1, by1, Wy2, by2,
           Lx1, bLx1, Lx2, bLx2, Lx3, bLx3,
           Ly1, bLy1, Ly2, bLy2, Ly3, bLy3):
  i32 = jnp.int32
  src_m = edge_index_m[0].astype(i32)
  dst_m = edge_index_m[1].astype(i32)
  src_d = edge_index_d[0].astype(i32)
  dst_d = edge_index_d[1].astype(i32)
  dst2_m = dst_m.reshape(E // EB, EB)
  dst2_d = dst_d.reshape(E // EB, EB)

  ew_m, degp_m = _prep(data_m.reshape(-1), src_m, dst2_m)
  ew_d, degp_d = _prep(data_d.reshape(-1), src_d, dst2_d)
  dis_m, dis_d = _dis_kernel(degp_m, degp_d)

  def one_graph(x, src, dst2, ew, dis, W1, b1, W2, b2, L1, bL1, L2, bL2, L3, bL3):
    A2d = _abuild(src, dst2, ew).reshape(N, N)
    u1 = _t1(x, W1, dis)
    u2 = _g1(A2d, u1, dis, b1.reshape(1, -1), W2)
    return _g2(A2d, u2, dis, b2.reshape(1, -1),
               L1, bL1.reshape(1, -1), L2, bL2.reshape(1, -1),
               L3, bL3.reshape(1, -1))

  x3m = one_graph(x_m, src_m, dst2_m, ew_m, dis_m,
                  Wx1, bx1, Wx2, bx2, Lx1, bLx1, Lx2, bLx2, Lx3, bLx3)
  x3d = one_graph(x_d, src_d, dst2_d, ew_d, dis_d,
                  Wy1, by1, Wy2, by2, Ly1, bLy1, Ly2, bLy2, Ly3, bLy3)
  return _final(x3m, x3d)


# speedup vs baseline: 3.5212x; 3.5212x over previous
import jax
import jax.numpy as jnp
from jax import lax
from jax.experimental import pallas as pl
from jax.experimental.pallas import tpu as pltpu
from jax.experimental.pallas import tpu_sc as plsc

N = 4096
F = 256
E = 131072
NC = 2
NS = 16
NW = NC * NS
EPW = E // NW
EB = 128
NB = EPW // EB
RPT = N // NS

_f32 = jnp.float32
_MESH = plsc.VectorSubcoreMesh(
    core_axis_name="c", subcore_axis_name="s", num_cores=NC, num_subcores=NS)


def _prep_body(data_hbm, src_hbm, dst2_hbm,
               ew_hbm, degp_hbm,
               src_v, dst2_v, idx_v, ew_v, z_v, deg_sh, sem):
  cid = lax.axis_index("c")
  sid = lax.axis_index("s")
  wid = sid * NC + cid
  ebase = wid * EPW
  pltpu.sync_copy(src_hbm.at[pl.ds(ebase, EPW)], src_v)
  pltpu.sync_copy(dst2_hbm.at[pl.ds(wid * NB, NB)], dst2_v)

  def idx_body(b, c):
    for j in range(EB // 16):
      s16 = src_v[pl.ds(b * EB + j * 16, 16)]
      d16 = dst2_v[b, pl.ds(j * 16, 16)]
      idx_v[pl.ds(b * EB + j * 16, 16)] = s16 * N + d16
    return c
  lax.fori_loop(0, NB, idx_body, 0)

  cps = []
  for b in range(NB):
    cps.append(pltpu.async_copy(
        data_hbm.at[idx_v.at[pl.ds(b * EB, EB)]],
        ew_v.at[pl.ds(b * EB, EB)], sem))
  for cp in cps:
    cp.wait()
  pltpu.sync_copy(ew_v, ew_hbm.at[pl.ds(ebase, EPW)])

  for k in range(RPT // 16):
    z_v[pl.ds(k * 16, 16)] = jnp.zeros((16,), _f32)
  pltpu.sync_copy(z_v, deg_sh.at[pl.ds(sid * RPT, RPT)])
  plsc.subcore_barrier()
  for b in range(NB):
    pltpu.sync_copy(ew_v.at[pl.ds(b * EB, EB)],
                    deg_sh.at[dst2_v.at[b]], add=True)
  plsc.subcore_barrier()
  pltpu.sync_copy(deg_sh.at[pl.ds(sid * RPT, RPT)],
                  degp_hbm.at[pl.ds(cid * N + sid * RPT, RPT)])


def _prep(data_flat, src, dst2):
  return pl.kernel(
      _prep_body,
      out_type=(jax.ShapeDtypeStruct((E,), _f32),
                jax.ShapeDtypeStruct((NC * N,), _f32)),
      mesh=_MESH,
      scratch_types=[
          pltpu.VMEM((EPW,), jnp.int32),
          pltpu.VMEM((NB, EB), jnp.int32),
          pltpu.VMEM((EPW,), jnp.int32),
          pltpu.VMEM((EPW,), _f32),
          pltpu.VMEM((RPT,), _f32),
          pltpu.VMEM_SHARED((N,), _f32),
          pltpu.SemaphoreType.DMA,
      ],
  )(data_flat, src, dst2)


EPT = E // NS
NB2 = EPT // EB
SR = 256
NSLAB = N // SR // NC
DUMP = SR * N
ZW = 4096


CAP = 768


def _abuild(src, dst2, ew):
  def body(src_hbm, dst2_hbm, ew_hbm, a_hbm,
           src_v, dst2_v, ew_v, gidx_v, idx_v, zb_v, slab_sh, sem):
    cid = lax.axis_index("c")
    sid = lax.axis_index("s")
    ebase = sid * EPT
    pltpu.sync_copy(src_hbm.at[pl.ds(ebase, EPT)], src_v)
    pltpu.sync_copy(dst2_hbm.at[pl.ds(sid * NB2, NB2)], dst2_v)
    pltpu.sync_copy(ew_hbm.at[pl.ds(ebase, EPT)], ew_v)

    def zb_body(i, c):
      zb_v[pl.ds(i * 16, 16)] = jnp.zeros((16,), _f32)
      return c
    lax.fori_loop(0, ZW // 16, zb_body, 0)

    base16 = jnp.full((16,), 1, jnp.int32) * (cid * NSLAB)

    def pre_body(b, c):
      for j in range(EB // 16):
        s16 = src_v[pl.ds(b * EB + j * 16, 16)]
        d16 = dst2_v[b, pl.ds(j * 16, 16)]
        gidx_v[pl.ds(b * EB + j * 16, 16)] = (
            (d16 & jnp.full((16,), SR - 1, jnp.int32)) * N + s16)
      return c
    lax.fori_loop(0, NB2, pre_body, 0)

    dump16 = jnp.full((16,), DUMP, jnp.int32)
    for slab in range(NSLAB):
      lo = (cid * NSLAB + slab) * SR
      slab16 = jnp.full((16,), slab, jnp.int32)
      for k in range(SR * N // NS // ZW):
        pltpu.sync_copy(zb_v, slab_sh.at[pl.ds(sid * (SR * N // NS)
                                               + k * ZW, ZW)])
      plsc.subcore_barrier()

      def idx_body(b, c):
        for j in range(EB // 16):
          g16 = gidx_v[pl.ds(b * EB + j * 16, 16)]
          d16 = dst2_v[b, pl.ds(j * 16, 16)]
          ok = (lax.shift_right_logical(d16, 8) - base16) == slab16
          idx_v[b, pl.ds(j * 16, 16)] = jnp.where(ok, g16, dump16)
        return c
      lax.fori_loop(0, NB2, idx_body, 0)
      for b in range(NB2):
        pltpu.sync_copy(
            ew_v.at[pl.ds(b * EB, EB)],
            slab_sh.at[plsc.Indices(idx_v.at[b], ignored_value=DUMP)],
            add=True)
      plsc.subcore_barrier()
      pltpu.sync_copy(
          slab_sh.at[pl.ds(sid * (SR // NS) * N, (SR // NS) * N)],
          a_hbm.at[pl.ds((lo + sid * (SR // NS)) * N, (SR // NS) * N)])
      plsc.subcore_barrier()

  return pl.kernel(
      body,
      out_type=jax.ShapeDtypeStruct((N * N,), _f32),
      mesh=_MESH,
      scratch_types=[
          pltpu.VMEM((EPT,), jnp.int32),
          pltpu.VMEM((NB2, EB), jnp.int32),
          pltpu.VMEM((EPT,), _f32),
          pltpu.VMEM((EPT,), jnp.int32),
          pltpu.VMEM((NB2, EB), jnp.int32),
          pltpu.VMEM((ZW,), _f32),
          pltpu.VMEM_SHARED((SR * N + 8,), _f32),
          pltpu.SemaphoreType.DMA,
      ],
  )(src, dst2, ew)


def _dis_kernel(degm, degd):
  R = N // 128

  def body(dm_ref, dd_ref, om_ref, od_ref):
    ones = jnp.ones((1, 128), _f32)
    for dref, oref in ((dm_ref, om_ref), (dd_ref, od_ref)):
      d = dref[0:R] + dref[R:2 * R] + 1.0
      ok = d > 0
      ds_ = jnp.where(ok, d, 1.0)
      y = lax.rsqrt(ds_)
      y = 0.5 * y * (3.0 - ds_ * y * y)
      y = 0.5 * y * (3.0 - ds_ * y * y)
      dis = jnp.where(ok, y, 0.0)
      for i in range(R):
        oref[pl.ds(i * 128, 128), :] = lax.dot_general(
            dis[i:i + 1, :], ones, (((0,), (0,)), ((), ())),
            preferred_element_type=_f32, precision=lax.Precision.HIGHEST)
  out = pl.pallas_call(
      body,
      out_shape=[jax.ShapeDtypeStruct((N, 128), _f32),
                 jax.ShapeDtypeStruct((N, 128), _f32)],
  )(degm.reshape(2 * R, 128), degd.reshape(2 * R, 128))
  return out


_BR = 512


def _t1(x, W, dis2d):
  def body(x_ref, w_ref, dis_ref, o_ref):
    xw = jnp.dot(x_ref[...], w_ref[...], preferred_element_type=_f32, precision=lax.Precision.HIGHEST)
    o_ref[...] = dis_ref[:, 0:1] * xw
  return pl.pallas_call(
      body,
      grid=(N // _BR,),
      in_specs=[pl.BlockSpec((_BR, F), lambda i: (i, 0)),
                pl.BlockSpec((F, F), lambda i: (0, 0)),
                pl.BlockSpec((_BR, 128), lambda i: (i, 0))],
      out_specs=pl.BlockSpec((_BR, F), lambda i: (i, 0)),
      out_shape=jax.ShapeDtypeStruct((N, F), _f32),
  )(x, W, dis2d)


def _g1(A2d, u, dis2d, b1, W2):
  def body(a_ref, u_ref, dis_ref, b_ref, w_ref, o_ref):
    i = pl.program_id(0)
    acc = jnp.dot(a_ref[...], u_ref[...], preferred_element_type=_f32, precision=lax.Precision.HIGHEST)
    u_blk = u_ref[pl.ds(i * _BR, _BR), :]
    dis = dis_ref[:, 0:1]
    h = jax.nn.relu(dis * (acc + u_blk) + b_ref[...])
    o_ref[...] = dis * jnp.dot(h, w_ref[...], preferred_element_type=_f32, precision=lax.Precision.HIGHEST)
  return pl.pallas_call(
      body,
      grid=(N // _BR,),
      in_specs=[pl.BlockSpec((_BR, N), lambda i: (i, 0)),
                pl.BlockSpec((N, F), lambda i: (0, 0)),
                pl.BlockSpec((_BR, 128), lambda i: (i, 0)),
                pl.BlockSpec((1, F), lambda i: (0, 0)),
                pl.BlockSpec((F, F), lambda i: (0, 0))],
      out_specs=pl.BlockSpec((_BR, F), lambda i: (i, 0)),
      out_shape=jax.ShapeDtypeStruct((N, F), _f32),
  )(A2d, u, dis2d, b1, W2)


def _g2(A2d, u, dis2d, b2, L1, bL1, L2, bL2, L3, bL3):
  def body(a_ref, u_ref, dis_ref, b_ref,
           l1_ref, b1_ref, l2_ref, b2_ref, l3_ref, b3_ref, o_ref):
    i = pl.program_id(0)
    acc = jnp.dot(a_ref[...], u_ref[...], preferred_element_type=_f32, precision=lax.Precision.HIGHEST)
    u_blk = u_ref[pl.ds(i * _BR, _BR), :]
    dis = dis_ref[:, 0:1]
    X = jax.nn.relu(dis * (acc + u_blk) + b_ref[...])
    x1 = jax.nn.relu(jnp.dot(X, l1_ref[...], preferred_element_type=_f32, precision=lax.Precision.HIGHEST)
                     + b1_ref[...])
    x2 = jax.nn.relu(jnp.dot(x1, l2_ref[...], preferred_element_type=_f32, precision=lax.Precision.HIGHEST)
                     + b2_ref[...])
    o_ref[...] = jax.nn.relu(
        jnp.dot(x2, l3_ref[...], preferred_element_type=_f32, precision=lax.Precision.HIGHEST) + b3_ref[...])
  return pl.pallas_call(
      body,
      grid=(N // _BR,),
      in_specs=[pl.BlockSpec((_BR, N), lambda i: (i, 0)),
                pl.BlockSpec((N, F), lambda i: (0, 0)),
                pl.BlockSpec((_BR, 128), lambda i: (i, 0)),
                pl.BlockSpec((1, F), lambda i: (0, 0)),
                pl.BlockSpec((F, F), lambda i: (0, 0)),
                pl.BlockSpec((1, F), lambda i: (0, 0)),
                pl.BlockSpec((F, 128), lambda i: (0, 0)),
                pl.BlockSpec((1, 128), lambda i: (0, 0)),
                pl.BlockSpec((128, 64), lambda i: (0, 0)),
                pl.BlockSpec((1, 64), lambda i: (0, 0))],
      out_specs=pl.BlockSpec((_BR, 64), lambda i: (i, 0)),
      out_shape=jax.ShapeDtypeStruct((N, 64), _f32),
  )(A2d, u, dis2d, b2, L1, bL1, L2, bL2, L3, bL3)


def _final(a, b):
  def body(a_ref, b_ref, o_ref):
    o_ref[...] = lax.dot_general(a_ref[...], b_ref[...],
                                 (((1,), (1,)), ((), ())),
                                 preferred_element_type=_f32, precision=lax.Precision.HIGHEST)
  return pl.pallas_call(
      body,
      grid=(N // _BR, N // _BR),
      in_specs=[pl.BlockSpec((_BR, 64), lambda i, j: (i, 0)),
                pl.BlockSpec((_BR, 64), lambda i, j: (j, 0))],
      out_specs=pl.BlockSpec((_BR, _BR), lambda i, j: (i, j)),
      out_shape=jax.ShapeDtypeStruct((N, N), _f32),
  )(a, b)


def kernel(x_m, x_d, data_m, data_d, edge_index_m, edge_index_d,
           Wx1, bx1, Wx2, bx2, Wy1, by1, Wy2, by2,
           Lx1, bLx1, Lx2, bLx2, Lx3, bLx3,
           Ly1, bLy1, Ly2, bLy2, Ly3, bLy3):
  i32 = jnp.int32
  src_m = edge_index_m[0].astype(i32)
  dst_m = edge_index_m[1].astype(i32)
  src_d = edge_index_d[0].astype(i32)
  dst_d = edge_index_d[1].astype(i32)
  dst2_m = dst_m.reshape(E // EB, EB)
  dst2_d = dst_d.reshape(E // EB, EB)

  ew_m, degp_m = _prep(data_m.reshape(-1), src_m, dst2_m)
  ew_d, degp_d = _prep(data_d.reshape(-1), src_d, dst2_d)
  dis_m, dis_d = _dis_kernel(degp_m, degp_d)

  def one_graph(x, src, dst2, ew, dis, W1, b1, W2, b2, L1, bL1, L2, bL2, L3, bL3):
    A2d = _abuild(src, dst2, ew).reshape(N, N)
    u1 = _t1(x, W1, dis)
    u2 = _g1(A2d, u1, dis, b1.reshape(1, -1), W2)
    return _g2(A2d, u2, dis, b2.reshape(1, -1),
               L1, bL1.reshape(1, -1), L2, bL2.reshape(1, -1),
               L3, bL3.reshape(1, -1))

  x3m = one_graph(x_m, src_m, dst2_m, ew_m, dis_m,
                  Wx1, bx1, Wx2, bx2, Lx1, bLx1, Lx2, bLx2, Lx3, bLx3)
  x3d = one_graph(x_d, src_d, dst2_d, ew_d, dis_d,
                  Wy1, by1, Wy2, by2, Ly1, bLy1, Ly2, bLy2, Ly3, bLy3)
  return _final(x3m, x3d)


# speedup vs baseline: 4.2111x; 1.1959x over previous
import jax
import jax.numpy as jnp
from jax import lax
from jax.experimental import pallas as pl
from jax.experimental.pallas import tpu as pltpu
from jax.experimental.pallas import tpu_sc as plsc

N = 4096
F = 256
E = 131072
NC = 2
NS = 16
NW = NC * NS
EPW = E // NW
EB = 128
NB = EPW // EB
RPT = N // NS

_f32 = jnp.float32
_MESH = plsc.VectorSubcoreMesh(
    core_axis_name="c", subcore_axis_name="s", num_cores=NC, num_subcores=NS)


def _prep_body(data_hbm, src_hbm, dst2_hbm,
               ew_hbm, degp_hbm,
               src_v, dst2_v, idx_v, ew_v, z_v, deg_sh, sem):
  cid = lax.axis_index("c")
  sid = lax.axis_index("s")
  wid = sid * NC + cid
  ebase = wid * EPW
  pltpu.sync_copy(src_hbm.at[pl.ds(ebase, EPW)], src_v)
  pltpu.sync_copy(dst2_hbm.at[pl.ds(wid * NB, NB)], dst2_v)

  def idx_body(b, c):
    for j in range(EB // 16):
      s16 = src_v[pl.ds(b * EB + j * 16, 16)]
      d16 = dst2_v[b, pl.ds(j * 16, 16)]
      idx_v[pl.ds(b * EB + j * 16, 16)] = s16 * N + d16
    return c
  lax.fori_loop(0, NB, idx_body, 0)

  cps = []
  for b in range(NB):
    cps.append(pltpu.async_copy(
        data_hbm.at[idx_v.at[pl.ds(b * EB, EB)]],
        ew_v.at[pl.ds(b * EB, EB)], sem))
  for cp in cps:
    cp.wait()
  pltpu.sync_copy(ew_v, ew_hbm.at[pl.ds(ebase, EPW)])

  for k in range(RPT // 16):
    z_v[pl.ds(k * 16, 16)] = jnp.zeros((16,), _f32)
  pltpu.sync_copy(z_v, deg_sh.at[pl.ds(sid * RPT, RPT)])
  plsc.subcore_barrier()
  for b in range(NB):
    pltpu.sync_copy(ew_v.at[pl.ds(b * EB, EB)],
                    deg_sh.at[dst2_v.at[b]], add=True)
  plsc.subcore_barrier()
  pltpu.sync_copy(deg_sh.at[pl.ds(sid * RPT, RPT)],
                  degp_hbm.at[pl.ds(cid * N + sid * RPT, RPT)])


def _prep(data_flat, src, dst2):
  return pl.kernel(
      _prep_body,
      out_type=(jax.ShapeDtypeStruct((E,), _f32),
                jax.ShapeDtypeStruct((NC * N,), _f32)),
      mesh=_MESH,
      scratch_types=[
          pltpu.VMEM((EPW,), jnp.int32),
          pltpu.VMEM((NB, EB), jnp.int32),
          pltpu.VMEM((EPW,), jnp.int32),
          pltpu.VMEM((EPW,), _f32),
          pltpu.VMEM((RPT,), _f32),
          pltpu.VMEM_SHARED((N,), _f32),
          pltpu.SemaphoreType.DMA,
      ],
  )(data_flat, src, dst2)


EPT = E // NS
NB2 = EPT // EB
SR = 256
NSLAB = N // SR // NC
DUMP = SR * N
ZW = 4096


CAP = 768


def _abuild(src, dst2, ew):
  def body(src_hbm, dst2_hbm, ew_hbm, a_hbm,
           src_v, dst2_v, ew_v, gidx_v, idx_v, zb_v, slab_sh, sem):
    cid = lax.axis_index("c")
    sid = lax.axis_index("s")
    ebase = sid * EPT
    pltpu.sync_copy(src_hbm.at[pl.ds(ebase, EPT)], src_v)
    pltpu.sync_copy(dst2_hbm.at[pl.ds(sid * NB2, NB2)], dst2_v)
    pltpu.sync_copy(ew_hbm.at[pl.ds(ebase, EPT)], ew_v)

    def zb_body(i, c):
      zb_v[pl.ds(i * 16, 16)] = jnp.zeros((16,), _f32)
      return c
    lax.fori_loop(0, ZW // 16, zb_body, 0)

    base16 = jnp.full((16,), 1, jnp.int32) * (cid * NSLAB)

    def pre_body(b, c):
      for j in range(EB // 16):
        s16 = src_v[pl.ds(b * EB + j * 16, 16)]
        d16 = dst2_v[b, pl.ds(j * 16, 16)]
        gidx_v[pl.ds(b * EB + j * 16, 16)] = (
            (d16 & jnp.full((16,), SR - 1, jnp.int32)) * N + s16)
      return c
    lax.fori_loop(0, NB2, pre_body, 0)

    dump16 = jnp.full((16,), DUMP, jnp.int32)
    for slab in range(NSLAB):
      lo = (cid * NSLAB + slab) * SR
      slab16 = jnp.full((16,), slab, jnp.int32)
      for k in range(SR * N // NS // ZW):
        pltpu.sync_copy(zb_v, slab_sh.at[pl.ds(sid * (SR * N // NS)
                                               + k * ZW, ZW)])
      plsc.subcore_barrier()

      def idx_body(b, c):
        for j in range(EB // 16):
          g16 = gidx_v[pl.ds(b * EB + j * 16, 16)]
          d16 = dst2_v[b, pl.ds(j * 16, 16)]
          ok = (lax.shift_right_logical(d16, 8) - base16) == slab16
          idx_v[b, pl.ds(j * 16, 16)] = jnp.where(ok, g16, dump16)
        return c
      lax.fori_loop(0, NB2, idx_body, 0)
      for b in range(NB2):
        pltpu.sync_copy(
            ew_v.at[pl.ds(b * EB, EB)],
            slab_sh.at[plsc.Indices(idx_v.at[b], ignored_value=DUMP)],
            add=True)
      plsc.subcore_barrier()
      pltpu.sync_copy(
          slab_sh.at[pl.ds(sid * (SR // NS) * N, (SR // NS) * N)],
          a_hbm.at[pl.ds((lo + sid * (SR // NS)) * N, (SR // NS) * N)])
      plsc.subcore_barrier()

  return pl.kernel(
      body,
      out_type=jax.ShapeDtypeStruct((N * N,), _f32),
      mesh=_MESH,
      scratch_types=[
          pltpu.VMEM((EPT,), jnp.int32),
          pltpu.VMEM((NB2, EB), jnp.int32),
          pltpu.VMEM((EPT,), _f32),
          pltpu.VMEM((EPT,), jnp.int32),
          pltpu.VMEM((NB2, EB), jnp.int32),
          pltpu.VMEM((ZW,), _f32),
          pltpu.VMEM_SHARED((SR * N + 8,), _f32),
          pltpu.SemaphoreType.DMA,
      ],
  )(src, dst2, ew)


def _dis_kernel(degm, degd):
  R = N // 128

  def body(dm_ref, dd_ref, om_ref, od_ref):
    ones = jnp.ones((1, 128), _f32)
    for dref, oref in ((dm_ref, om_ref), (dd_ref, od_ref)):
      d = dref[0:R] + dref[R:2 * R] + 1.0
      ok = d > 0
      ds_ = jnp.where(ok, d, 1.0)
      y = lax.rsqrt(ds_)
      y = 0.5 * y * (3.0 - ds_ * y * y)
      y = 0.5 * y * (3.0 - ds_ * y * y)
      dis = jnp.where(ok, y, 0.0)
      for i in range(R):
        oref[pl.ds(i * 128, 128), :] = lax.dot_general(
            dis[i:i + 1, :], ones, (((0,), (0,)), ((), ())),
            preferred_element_type=_f32, precision=lax.Precision.HIGHEST)
  out = pl.pallas_call(
      body,
      out_shape=[jax.ShapeDtypeStruct((N, 128), _f32),
                 jax.ShapeDtypeStruct((N, 128), _f32)],
  )(degm.reshape(2 * R, 128), degd.reshape(2 * R, 128))
  return out


_BR = 512


def _t1(x, W, dis2d):
  def body(x_ref, w_ref, dis_ref, o_ref):
    xw = jnp.dot(x_ref[...], w_ref[...], preferred_element_type=_f32, precision=lax.Precision.HIGHEST)
    o_ref[...] = dis_ref[:, 0:1] * xw
  return pl.pallas_call(
      body,
      grid=(N // _BR,),
      in_specs=[pl.BlockSpec((_BR, F), lambda i: (i, 0)),
                pl.BlockSpec((F, F), lambda i: (0, 0)),
                pl.BlockSpec((_BR, 128), lambda i: (i, 0))],
      out_specs=pl.BlockSpec((_BR, F), lambda i: (i, 0)),
      out_shape=jax.ShapeDtypeStruct((N, F), _f32),
  )(x, W, dis2d)


def _g1(A2d, u, dis2d, b1, W2):
  def body(a_ref, u_ref, dis_ref, b_ref, w_ref, o_ref):
    i = pl.program_id(0)
    acc = jnp.dot(a_ref[...], u_ref[...], preferred_element_type=_f32)
    u_blk = u_ref[pl.ds(i * _BR, _BR), :]
    dis = dis_ref[:, 0:1]
    h = jax.nn.relu(dis * (acc + u_blk) + b_ref[...])
    o_ref[...] = dis * jnp.dot(h, w_ref[...], preferred_element_type=_f32, precision=lax.Precision.HIGHEST)
  return pl.pallas_call(
      body,
      grid=(N // _BR,),
      in_specs=[pl.BlockSpec((_BR, N), lambda i: (i, 0)),
                pl.BlockSpec((N, F), lambda i: (0, 0)),
                pl.BlockSpec((_BR, 128), lambda i: (i, 0)),
                pl.BlockSpec((1, F), lambda i: (0, 0)),
                pl.BlockSpec((F, F), lambda i: (0, 0))],
      out_specs=pl.BlockSpec((_BR, F), lambda i: (i, 0)),
      out_shape=jax.ShapeDtypeStruct((N, F), _f32),
  )(A2d, u, dis2d, b1, W2)


def _g2(A2d, u, dis2d, b2, L1, bL1, L2, bL2, L3, bL3):
  def body(a_ref, u_ref, dis_ref, b_ref,
           l1_ref, b1_ref, l2_ref, b2_ref, l3_ref, b3_ref, o_ref):
    i = pl.program_id(0)
    acc = jnp.dot(a_ref[...], u_ref[...], preferred_element_type=_f32)
    u_blk = u_ref[pl.ds(i * _BR, _BR), :]
    dis = dis_ref[:, 0:1]
    X = jax.nn.relu(dis * (acc + u_blk) + b_ref[...])
    x1 = jax.nn.relu(jnp.dot(X, l1_ref[...], preferred_element_type=_f32, precision=lax.Precision.HIGHEST)
                     + b1_ref[...])
    x2 = jax.nn.relu(jnp.dot(x1, l2_ref[...], preferred_element_type=_f32, precision=lax.Precision.HIGHEST)
                     + b2_ref[...])
    o_ref[...] = jax.nn.relu(
        jnp.dot(x2, l3_ref[...], preferred_element_type=_f32, precision=lax.Precision.HIGHEST) + b3_ref[...])
  return pl.pallas_call(
      body,
      grid=(N // _BR,),
      in_specs=[pl.BlockSpec((_BR, N), lambda i: (i, 0)),
                pl.BlockSpec((N, F), lambda i: (0, 0)),
                pl.BlockSpec((_BR, 128), lambda i: (i, 0)),
                pl.BlockSpec((1, F), lambda i: (0, 0)),
                pl.BlockSpec((F, F), lambda i: (0, 0)),
                pl.BlockSpec((1, F), lambda i: (0, 0)),
                pl.BlockSpec((F, 128), lambda i: (0, 0)),
                pl.BlockSpec((1, 128), lambda i: (0, 0)),
                pl.BlockSpec((128, 64), lambda i: (0, 0)),
                pl.BlockSpec((1, 64), lambda i: (0, 0))],
      out_specs=pl.BlockSpec((_BR, 64), lambda i: (i, 0)),
      out_shape=jax.ShapeDtypeStruct((N, 64), _f32),
  )(A2d, u, dis2d, b2, L1, bL1, L2, bL2, L3, bL3)


def _final(a, b):
  def body(a_ref, b_ref, o_ref):
    o_ref[...] = lax.dot_general(a_ref[...], b_ref[...],
                                 (((1,), (1,)), ((), ())),
                                 preferred_element_type=_f32, precision=lax.Precision.HIGHEST)
  return pl.pallas_call(
      body,
      grid=(N // _BR, N // _BR),
      in_specs=[pl.BlockSpec((_BR, 64), lambda i, j: (i, 0)),
                pl.BlockSpec((_BR, 64), lambda i, j: (j, 0))],
      out_specs=pl.BlockSpec((_BR, _BR), lambda i, j: (i, j)),
      out_shape=jax.ShapeDtypeStruct((N, N), _f32),
  )(a, b)


def kernel(x_m, x_d, data_m, data_d, edge_index_m, edge_index_d,
           Wx1, bx1, Wx2, bx2, Wy1, by1, Wy2, by2,
           Lx1, bLx1, Lx2, bLx2, Lx3, bLx3,
           Ly1, bLy1, Ly2, bLy2, Ly3, bLy3):
  i32 = jnp.int32
  src_m = edge_index_m[0].astype(i32)
  dst_m = edge_index_m[1].astype(i32)
  src_d = edge_index_d[0].astype(i32)
  dst_d = edge_index_d[1].astype(i32)
  dst2_m = dst_m.reshape(E // EB, EB)
  dst2_d = dst_d.reshape(E // EB, EB)

  ew_m, degp_m = _prep(data_m.reshape(-1), src_m, dst2_m)
  ew_d, degp_d = _prep(data_d.reshape(-1), src_d, dst2_d)
  dis_m, dis_d = _dis_kernel(degp_m, degp_d)

  def one_graph(x, src, dst2, ew, dis, W1, b1, W2, b2, L1, bL1, L2, bL2, L3, bL3):
    A2d = _abuild(src, dst2, ew).reshape(N, N)
    u1 = _t1(x, W1, dis)
    u2 = _g1(A2d, u1, dis, b1.reshape(1, -1), W2)
    return _g2(A2d, u2, dis, b2.reshape(1, -1),
               L1, bL1.reshape(1, -1), L2, bL2.reshape(1, -1),
               L3, bL3.reshape(1, -1))

  x3m = one_graph(x_m, src_m, dst2_m, ew_m, dis_m,
                  Wx1, bx1, Wx2, bx2, Lx1, bLx1, Lx2, bLx2, Lx3, bLx3)
  x3d = one_graph(x_d, src_d, dst2_d, ew_d, dis_d,
                  Wy1, by1, Wy2, by2, Ly1, bLy1, Ly2, bLy2, Ly3, bLy3)
  return _final(x3m, x3d)


# speedup vs baseline: 4.5302x; 1.0758x over previous
import jax
import jax.numpy as jnp
from jax import lax
from jax.experimental import pallas as pl
from jax.experimental.pallas import tpu as pltpu
from jax.experimental.pallas import tpu_sc as plsc

N = 4096
F = 256
E = 131072
NC = 2
NS = 16
NW = NC * NS
EPW = E // NW
EB = 128
NB = EPW // EB
RPT = N // NS

_f32 = jnp.float32
_MESH = plsc.VectorSubcoreMesh(
    core_axis_name="c", subcore_axis_name="s", num_cores=NC, num_subcores=NS)


def _prep_body(data_hbm, src_hbm, dst2_hbm,
               ew_hbm, degp_hbm,
               src_v, dst2_v, idx_v, ew_v, z_v, deg_sh, sem):
  cid = lax.axis_index("c")
  sid = lax.axis_index("s")
  wid = sid * NC + cid
  ebase = wid * EPW
  pltpu.sync_copy(src_hbm.at[pl.ds(ebase, EPW)], src_v)
  pltpu.sync_copy(dst2_hbm.at[pl.ds(wid * NB, NB)], dst2_v)

  def idx_body(b, c):
    for j in range(EB // 16):
      s16 = src_v[pl.ds(b * EB + j * 16, 16)]
      d16 = dst2_v[b, pl.ds(j * 16, 16)]
      idx_v[pl.ds(b * EB + j * 16, 16)] = s16 * N + d16
    return c
  lax.fori_loop(0, NB, idx_body, 0)

  cps = []
  for b in range(NB):
    cps.append(pltpu.async_copy(
        data_hbm.at[idx_v.at[pl.ds(b * EB, EB)]],
        ew_v.at[pl.ds(b * EB, EB)], sem))
  for cp in cps:
    cp.wait()
  pltpu.sync_copy(ew_v, ew_hbm.at[pl.ds(ebase, EPW)])

  for k in range(RPT // 16):
    z_v[pl.ds(k * 16, 16)] = jnp.zeros((16,), _f32)
  pltpu.sync_copy(z_v, deg_sh.at[pl.ds(sid * RPT, RPT)])
  plsc.subcore_barrier()
  for b in range(NB):
    pltpu.sync_copy(ew_v.at[pl.ds(b * EB, EB)],
                    deg_sh.at[dst2_v.at[b]], add=True)
  plsc.subcore_barrier()
  pltpu.sync_copy(deg_sh.at[pl.ds(sid * RPT, RPT)],
                  degp_hbm.at[pl.ds(cid * N + sid * RPT, RPT)])


def _prep(data_flat, src, dst2):
  return pl.kernel(
      _prep_body,
      out_type=(jax.ShapeDtypeStruct((E,), _f32),
                jax.ShapeDtypeStruct((NC * N,), _f32)),
      mesh=_MESH,
      scratch_types=[
          pltpu.VMEM((EPW,), jnp.int32),
          pltpu.VMEM((NB, EB), jnp.int32),
          pltpu.VMEM((EPW,), jnp.int32),
          pltpu.VMEM((EPW,), _f32),
          pltpu.VMEM((RPT,), _f32),
          pltpu.VMEM_SHARED((N,), _f32),
          pltpu.SemaphoreType.DMA,
      ],
  )(data_flat, src, dst2)


EPT = E // NS
NB2 = EPT // EB
SR = 256
NSLAB = N // SR // NC
DUMP = SR * N
ZW = 4096


CAP = 768


def _abuild(src, dst2, ew):
  def body(src_hbm, dst2_hbm, ew_hbm, a_hbm,
           src_v, dst2_v, ew_v, gidx_v, idx_v, zb_v, slab_sh, sem):
    cid = lax.axis_index("c")
    sid = lax.axis_index("s")
    ebase = sid * EPT
    pltpu.sync_copy(src_hbm.at[pl.ds(ebase, EPT)], src_v)
    pltpu.sync_copy(dst2_hbm.at[pl.ds(sid * NB2, NB2)], dst2_v)
    pltpu.sync_copy(ew_hbm.at[pl.ds(ebase, EPT)], ew_v)

    def zb_body(i, c):
      zb_v[pl.ds(i * 16, 16)] = jnp.zeros((16,), _f32)
      return c
    lax.fori_loop(0, ZW // 16, zb_body, 0)

    base16 = jnp.full((16,), 1, jnp.int32) * (cid * NSLAB)

    def pre_body(b, c):
      for j in range(EB // 16):
        s16 = src_v[pl.ds(b * EB + j * 16, 16)]
        d16 = dst2_v[b, pl.ds(j * 16, 16)]
        gidx_v[pl.ds(b * EB + j * 16, 16)] = (
            (d16 & jnp.full((16,), SR - 1, jnp.int32)) * N + s16)
      return c
    lax.fori_loop(0, NB2, pre_body, 0)

    dump16 = jnp.full((16,), DUMP, jnp.int32)
    for slab in range(NSLAB):
      lo = (cid * NSLAB + slab) * SR
      slab16 = jnp.full((16,), slab, jnp.int32)
      for k in range(SR * N // NS // ZW):
        pltpu.sync_copy(zb_v, slab_sh.at[pl.ds(sid * (SR * N // NS)
                                               + k * ZW, ZW)])
      plsc.subcore_barrier()

      def idx_body(b, c):
        for j in range(EB // 16):
          g16 = gidx_v[pl.ds(b * EB + j * 16, 16)]
          d16 = dst2_v[b, pl.ds(j * 16, 16)]
          ok = (lax.shift_right_logical(d16, 8) - base16) == slab16
          idx_v[b, pl.ds(j * 16, 16)] = jnp.where(ok, g16, dump16)
        return c
      lax.fori_loop(0, NB2, idx_body, 0)
      for b in range(NB2):
        pltpu.sync_copy(
            ew_v.at[pl.ds(b * EB, EB)],
            slab_sh.at[plsc.Indices(idx_v.at[b], ignored_value=DUMP)],
            add=True)
      plsc.subcore_barrier()
      pltpu.sync_copy(
          slab_sh.at[pl.ds(sid * (SR // NS) * N, (SR // NS) * N)],
          a_hbm.at[pl.ds((lo + sid * (SR // NS)) * N, (SR // NS) * N)])
      plsc.subcore_barrier()

  return pl.kernel(
      body,
      out_type=jax.ShapeDtypeStruct((N * N,), _f32),
      mesh=_MESH,
      scratch_types=[
          pltpu.VMEM((EPT,), jnp.int32),
          pltpu.VMEM((NB2, EB), jnp.int32),
          pltpu.VMEM((EPT,), _f32),
          pltpu.VMEM((EPT,), jnp.int32),
          pltpu.VMEM((NB2, EB), jnp.int32),
          pltpu.VMEM((ZW,), _f32),
          pltpu.VMEM_SHARED((SR * N + 8,), _f32),
          pltpu.SemaphoreType.DMA,
      ],
  )(src, dst2, ew)


def _dis_kernel(degm, degd):
  R = N // 128

  def body(dm_ref, dd_ref, om_ref, od_ref):
    ones = jnp.ones((1, 128), _f32)
    for dref, oref in ((dm_ref, om_ref), (dd_ref, od_ref)):
      d = dref[0:R] + dref[R:2 * R] + 1.0
      ok = d > 0
      ds_ = jnp.where(ok, d, 1.0)
      y = lax.rsqrt(ds_)
      y = 0.5 * y * (3.0 - ds_ * y * y)
      y = 0.5 * y * (3.0 - ds_ * y * y)
      dis = jnp.where(ok, y, 0.0)
      for i in range(R):
        oref[pl.ds(i * 128, 128), :] = lax.dot_general(
            dis[i:i + 1, :], ones, (((0,), (0,)), ((), ())),
            preferred_element_type=_f32, precision=lax.Precision.HIGHEST)
  out = pl.pallas_call(
      body,
      out_shape=[jax.ShapeDtypeStruct((N, 128), _f32),
                 jax.ShapeDtypeStruct((N, 128), _f32)],
  )(degm.reshape(2 * R, 128), degd.reshape(2 * R, 128))
  return out


_BR = 512


def _t1(x, W, dis2d):
  def body(x_ref, w_ref, dis_ref, o_ref):
    xw = jnp.dot(x_ref[...], w_ref[...], preferred_element_type=_f32)
    o_ref[...] = dis_ref[:, 0:1] * xw
  return pl.pallas_call(
      body,
      grid=(N // _BR,),
      in_specs=[pl.BlockSpec((_BR, F), lambda i: (i, 0)),
                pl.BlockSpec((F, F), lambda i: (0, 0)),
                pl.BlockSpec((_BR, 128), lambda i: (i, 0))],
      out_specs=pl.BlockSpec((_BR, F), lambda i: (i, 0)),
      out_shape=jax.ShapeDtypeStruct((N, F), _f32),
  )(x, W, dis2d)


def _g1(A2d, u, dis2d, b1, W2):
  def body(a_ref, u_ref, dis_ref, b_ref, w_ref, o_ref):
    i = pl.program_id(0)
    acc = jnp.dot(a_ref[...], u_ref[...], preferred_element_type=_f32)
    u_blk = u_ref[pl.ds(i * _BR, _BR), :]
    dis = dis_ref[:, 0:1]
    h = jax.nn.relu(dis * (acc + u_blk) + b_ref[...])
    o_ref[...] = dis * jnp.dot(h, w_ref[...], preferred_element_type=_f32)
  return pl.pallas_call(
      body,
      grid=(N // _BR,),
      in_specs=[pl.BlockSpec((_BR, N), lambda i: (i, 0)),
                pl.BlockSpec((N, F), lambda i: (0, 0)),
                pl.BlockSpec((_BR, 128), lambda i: (i, 0)),
                pl.BlockSpec((1, F), lambda i: (0, 0)),
                pl.BlockSpec((F, F), lambda i: (0, 0))],
      out_specs=pl.BlockSpec((_BR, F), lambda i: (i, 0)),
      out_shape=jax.ShapeDtypeStruct((N, F), _f32),
  )(A2d, u, dis2d, b1, W2)


def _g2(A2d, u, dis2d, b2, L1, bL1, L2, bL2, L3, bL3):
  def body(a_ref, u_ref, dis_ref, b_ref,
           l1_ref, b1_ref, l2_ref, b2_ref, l3_ref, b3_ref, o_ref):
    i = pl.program_id(0)
    acc = jnp.dot(a_ref[...], u_ref[...], preferred_element_type=_f32)
    u_blk = u_ref[pl.ds(i * _BR, _BR), :]
    dis = dis_ref[:, 0:1]
    X = jax.nn.relu(dis * (acc + u_blk) + b_ref[...])
    x1 = jax.nn.relu(jnp.dot(X, l1_ref[...], preferred_element_type=_f32)
                     + b1_ref[...])
    x2 = jax.nn.relu(jnp.dot(x1, l2_ref[...], preferred_element_type=_f32)
                     + b2_ref[...])
    o_ref[...] = jax.nn.relu(
        jnp.dot(x2, l3_ref[...], preferred_element_type=_f32) + b3_ref[...])
  return pl.pallas_call(
      body,
      grid=(N // _BR,),
      in_specs=[pl.BlockSpec((_BR, N), lambda i: (i, 0)),
                pl.BlockSpec((N, F), lambda i: (0, 0)),
                pl.BlockSpec((_BR, 128), lambda i: (i, 0)),
                pl.BlockSpec((1, F), lambda i: (0, 0)),
                pl.BlockSpec((F, F), lambda i: (0, 0)),
                pl.BlockSpec((1, F), lambda i: (0, 0)),
                pl.BlockSpec((F, 128), lambda i: (0, 0)),
                pl.BlockSpec((1, 128), lambda i: (0, 0)),
                pl.BlockSpec((128, 64), lambda i: (0, 0)),
                pl.BlockSpec((1, 64), lambda i: (0, 0))],
      out_specs=pl.BlockSpec((_BR, 64), lambda i: (i, 0)),
      out_shape=jax.ShapeDtypeStruct((N, 64), _f32),
  )(A2d, u, dis2d, b2, L1, bL1, L2, bL2, L3, bL3)


def _final(a, b):
  def body(a_ref, b_ref, o_ref):
    o_ref[...] = lax.dot_general(a_ref[...], b_ref[...],
                                 (((1,), (1,)), ((), ())),
                                 preferred_element_type=_f32)
  return pl.pallas_call(
      body,
      grid=(N // _BR, N // _BR),
      in_specs=[pl.BlockSpec((_BR, 64), lambda i, j: (i, 0)),
                pl.BlockSpec((_BR, 64), lambda i, j: (j, 0))],
      out_specs=pl.BlockSpec((_BR, _BR), lambda i, j: (i, j)),
      out_shape=jax.ShapeDtypeStruct((N, N), _f32),
  )(a, b)


def kernel(x_m, x_d, data_m, data_d, edge_index_m, edge_index_d,
           Wx1, bx1, Wx2, bx2, Wy1, by1, Wy2, by2,
           Lx1, bLx1, Lx2, bLx2, Lx3, bLx3,
           Ly1, bLy1, Ly2, bLy2, Ly3, bLy3):
  i32 = jnp.int32
  src_m = edge_index_m[0].astype(i32)
  dst_m = edge_index_m[1].astype(i32)
  src_d = edge_index_d[0].astype(i32)
  dst_d = edge_index_d[1].astype(i32)
  dst2_m = dst_m.reshape(E // EB, EB)
  dst2_d = dst_d.reshape(E // EB, EB)

  ew_m, degp_m = _prep(data_m.reshape(-1), src_m, dst2_m)
  ew_d, degp_d = _prep(data_d.reshape(-1), src_d, dst2_d)
  dis_m, dis_d = _dis_kernel(degp_m, degp_d)

  def one_graph(x, src, dst2, ew, dis, W1, b1, W2, b2, L1, bL1, L2, bL2, L3, bL3):
    A2d = _abuild(src, dst2, ew).reshape(N, N)
    u1 = _t1(x, W1, dis)
    u2 = _g1(A2d, u1, dis, b1.reshape(1, -1), W2)
    return _g2(A2d, u2, dis, b2.reshape(1, -1),
               L1, bL1.reshape(1, -1), L2, bL2.reshape(1, -1),
               L3, bL3.reshape(1, -1))

  x3m = one_graph(x_m, src_m, dst2_m, ew_m, dis_m,
                  Wx1, bx1, Wx2, bx2, Lx1, bLx1, Lx2, bLx2, Lx3, bLx3)
  x3d = one_graph(x_d, src_d, dst2_d, ew_d, dis_d,
                  Wy1, by1, Wy2, by2, Ly1, bLy1, Ly2, bLy2, Ly3, bLy3)
  return _final(x3m, x3d)
